# Initial kernel scaffold; baseline (speedup 1.0000x reference)
import jax
import jax.numpy as jnp
from jax.experimental import pallas as pl


def kernel(x, edge_index, noise, W1, b1, W2, b2, W3, b3):
    raise NotImplementedError("write your pallas kernel here")


# speedup vs baseline: 2.4944x; 2.4944x over previous
import functools

import jax
import jax.numpy as jnp
from jax import lax
from jax.experimental import pallas as pl
from jax.experimental.pallas import tpu as pltpu
from jax.experimental.pallas import tpu_sc as plsc

N_NODES = 10000
N_EDGES = 320000
IN_DIM = 128
HID_DIM = 128
OUT_DIM = 64
N_GRAPHS = 20
NPG = 500

NC = 2
NS = 16
CHUNK = 80
EROWS = N_NODES
SROWS = EROWS // NS
SVREG = SROWS * 2

QR = 500
NR = 10
CAP = 20000 + NR * CHUNK
TRASH = CAP + 64
STAG = CAP + 96
QMUL = 67109
QSHIFT = 25

CROWS = 4000
CNR = 3
CDUMP = 8


@functools.partial(
    pl.kernel,
    mesh=plsc.VectorSubcoreMesh(core_axis_name="c", subcore_axis_name="s"),
    out_type=jax.ShapeDtypeStruct((NC * CNR * 8, 500, 16), jnp.float32),
    scratch_types=[
        pltpu.VMEM((SROWS, 32), jnp.int32),
        pltpu.VMEM((SVREG // 5, CHUNK), jnp.int32),
        pltpu.VMEM((CHUNK, 16), jnp.float32),
        pltpu.VMEM((251, 16), jnp.float32),
        pltpu.VMEM_SHARED((CROWS + CDUMP, 16), jnp.float32),
    ],
    compiler_params=pltpu.CompilerParams(needs_layout_passes=False,
                                         use_tc_tiling_on_sc=False),
)
def _sc_counts(src_hbm, dst_hbm, out_hbm, ibuf, scat, ones_v, zbuf, acc):
    c = lax.axis_index("c")
    s = lax.axis_index("s")

    one16 = jnp.ones((16,), jnp.float32)
    zero16 = jnp.zeros((16,), jnp.float32)
    dump16 = CROWS + (lax.iota(jnp.int32, 16) & (CDUMP - 1))

    def fill_ones(i, _):
        ones_v[i, :] = one16
        return 0
    lax.fori_loop(0, CHUNK, fill_ones, 0)

    def fill_zero(i, _):
        zbuf[i, :] = zero16
        return 0
    lax.fori_loop(0, 251, fill_zero, 0)

    @pl.when(c == 0)
    def _():
        pltpu.sync_copy(src_hbm.at[pl.ds(s * SROWS, SROWS)], ibuf)

    @pl.when(c == 1)
    def _():
        pltpu.sync_copy(dst_hbm.at[pl.ds(s * SROWS, SROWS)], ibuf)

    for r in range(CNR):
        @pl.when(s < 8)
        def _():
            pltpu.sync_copy(zbuf, acc.at[pl.ds(s * 251, 251)])

        @pl.when(s >= 8)
        def _():
            pltpu.sync_copy(zbuf.at[pl.ds(0, 250)],
                            acc.at[pl.ds(2008 + (s - 8) * 250, 250)])

        def remap2(j, _):
            for k in range(5):
                i = j * 5 + k
                v = ibuf[i // 2, pl.ds((i % 2) * 16, 16)]
                lv = v - r * CROWS
                oob = (lv < 0) | (lv >= CROWS)
                scat[j, pl.ds(k * 16, 16)] = jnp.where(oob, dump16, lv)
            return 0
        lax.fori_loop(0, SVREG // 5, remap2, 0)
        plsc.subcore_barrier()

        def body(j, _):
            pltpu.sync_copy(ones_v, acc.at[scat.at[j]], add=True)
            return 0
        lax.fori_loop(0, SVREG // 5, body, 0)
        plsc.subcore_barrier()

        @pl.when(s < 8)
        def _():
            pltpu.sync_copy(acc.at[pl.ds(s * 500, 500)],
                            out_hbm.at[(c * CNR + r) * 8 + s])
        plsc.subcore_barrier()


def _partition(ibuf, dbuf, slist, dlist, c):
    iota16 = lax.iota(jnp.int32, 16)
    trash = TRASH + iota16
    half = c * 5000

    cnts = []
    for k in range(NR):
        lo_k = half + k * QR

        def count_body(i, cnt):
            dv = dbuf[i // 2, pl.ds((i % 2) * 16, 16)]
            m = (dv >= lo_k) & (dv < lo_k + QR)
            return cnt + jnp.sum(jnp.where(m, jnp.int32(1), jnp.int32(0)))
        cnts.append(lax.fori_loop(0, SVREG, count_body, jnp.int32(0)))

    nchunks = [cnts[k] // CHUNK + 1 for k in range(NR)]
    offs = []
    o = jnp.int32(0)
    for k in range(NR):
        offs.append(o * CHUNK)
        o = o + nchunks[k]

    curs = []
    for k in range(NR):
        lo_k = half + k * QR

        def place_body(i, cur):
            sv = ibuf[i // 2, pl.ds((i % 2) * 16, 16)]
            dv = dbuf[i // 2, pl.ds((i % 2) * 16, 16)]
            m = (dv >= lo_k) & (dv < lo_k + QR)
            mi = jnp.where(m, jnp.int32(1), jnp.int32(0))
            pos = jnp.where(m, cur + plsc.cumsum(mi) - 1, trash)
            plsc.store_scatter(slist, [pos], sv)
            plsc.store_scatter(dlist, [pos], dv - lo_k)
            return cur + jnp.sum(mi)
        curs.append(lax.fori_loop(0, SVREG, place_body, offs[k]))

    dumploc = jnp.full((16,), QR, jnp.int32)
    for k in range(NR):
        bound = offs[k] + nchunks[k] * CHUNK
        for t in range(CHUNK // 16 + 1):
            pos = curs[k] + t * 16 + iota16
            pos = jnp.where(pos < bound, pos, trash)
            plsc.store_scatter(slist, [pos], iota16)
            plsc.store_scatter(dlist, [pos], dumploc)
    return offs, nchunks


def _agg_round(r, s, offs, nchunks, flush, acc, rows, writeout):
    @pl.when(s < 15)
    def _():
        pltpu.sync_copy(rows.at[pl.ds(0, 31)], acc.at[pl.ds(s * 31, 31)])

    @pl.when(s == 15)
    def _():
        pltpu.sync_copy(rows.at[pl.ds(0, 36)], acc.at[pl.ds(465, 36)])
    plsc.subcore_barrier()

    def fl(j, _):
        flush(j, r)
        return 0
    lax.fori_loop(0, nchunks[r], fl, 0)
    plsc.subcore_barrier()

    writeout(r)
    plsc.subcore_barrier()


@functools.partial(
    pl.kernel,
    mesh=plsc.VectorSubcoreMesh(core_axis_name="c", subcore_axis_name="s"),
    out_type=jax.ShapeDtypeStruct((NC * NR * 4, 125, IN_DIM), jnp.float32),
    scratch_types=[
        pltpu.VMEM((SROWS, 32), jnp.int32),
        pltpu.VMEM((SROWS, 32), jnp.int32),
        pltpu.VMEM((STAG,), jnp.int32),
        pltpu.VMEM((STAG,), jnp.int32),
        pltpu.VMEM((CHUNK,), jnp.int32),
        pltpu.VMEM((CHUNK, IN_DIM), jnp.float32),
        pltpu.VMEM_SHARED((QR + 1, IN_DIM), jnp.float32),
        pltpu.SemaphoreType.DMA,
    ],
    compiler_params=pltpu.CompilerParams(needs_layout_passes=False,
                                         use_tc_tiling_on_sc=False),
)
def _sc_agg(feat_hbm, src_hbm, dst_hbm, out_hbm,
            ibuf, dbuf, slist, dlist, dstage, rows, acc, gsem):
    c = lax.axis_index("c")
    s = lax.axis_index("s")

    zero16 = jnp.zeros((16,), jnp.float32)

    def fill_zero(i, _):
        for kk in range(IN_DIM // 16):
            rows[i, pl.ds(kk * 16, 16)] = zero16
        return 0
    lax.fori_loop(0, 36, fill_zero, 0)

    pltpu.sync_copy(src_hbm.at[pl.ds(s * SROWS, SROWS)], ibuf)
    pltpu.sync_copy(dst_hbm.at[pl.ds(s * SROWS, SROWS)], dbuf)

    offs, nchunks = _partition(ibuf, dbuf, slist, dlist, c)

    def flush(j, r):
        base = offs[r] + j * CHUNK
        h = pltpu.async_copy(feat_hbm.at[slist.at[pl.ds(base, CHUNK)]],
                             rows, gsem)
        for k in range(CHUNK // 16):
            dstage[pl.ds(k * 16, 16)] = dlist[pl.ds(base + k * 16, 16)]
        h.wait()
        pltpu.sync_copy(rows, acc.at[dstage], add=True)

    def writeout(r):
        @pl.when(s < 4)
        def _():
            pltpu.sync_copy(acc.at[pl.ds(s * 125, 125)],
                            out_hbm.at[(c * NR + r) * 4 + s])

    for r in range(NR):
        _agg_round(r, s, offs, nchunks, flush, acc, rows, writeout)
        lax.fori_loop(0, 36, fill_zero, 0)


@functools.partial(
    pl.kernel,
    mesh=plsc.VectorSubcoreMesh(core_axis_name="c", subcore_axis_name="s"),
    out_type=[
        jax.ShapeDtypeStruct((NC * NR * 4, 125, OUT_DIM), jnp.float32),
        jax.ShapeDtypeStruct((NC * NR * 4, 125, OUT_DIM), jnp.float32),
        jax.ShapeDtypeStruct((NC * NR * 4, 125, OUT_DIM), jnp.float32),
    ],
    scratch_types=[
        pltpu.VMEM((SROWS, 32), jnp.int32),
        pltpu.VMEM((SROWS, 32), jnp.int32),
        pltpu.VMEM((STAG,), jnp.int32),
        pltpu.VMEM((STAG,), jnp.int32),
        pltpu.VMEM((CHUNK,), jnp.int32),
        pltpu.VMEM((CHUNK, IN_DIM), jnp.float32),
        pltpu.VMEM_SHARED((QR + 1, IN_DIM), jnp.float32),
        pltpu.VMEM((136, OUT_DIM), jnp.float32),
        pltpu.VMEM((136, OUT_DIM), jnp.float32),
        pltpu.VMEM((8, 16), jnp.float32),
        pltpu.SemaphoreType.DMA,
    ],
    compiler_params=pltpu.CompilerParams(needs_layout_passes=False,
                                         use_tc_tiling_on_sc=False),
)
def _sc_aggfin(feat_hbm, src_hbm, dst_hbm, ndw_hbm, noise_hbm, b23_hbm,
               mean_hbm, ls_hbm, z_hbm, ibuf, dbuf, slist, dlist,
               dstage, rows, acc, ndbuf, nzbuf, bbuf, gsem):
    c = lax.axis_index("c")
    s = lax.axis_index("s")

    zero16 = jnp.zeros((16,), jnp.float32)

    def fill_zero(i, _):
        for kk in range(IN_DIM // 16):
            rows[i, pl.ds(kk * 16, 16)] = zero16
        return 0
    lax.fori_loop(0, 36, fill_zero, 0)

    pltpu.sync_copy(src_hbm.at[pl.ds(s * SROWS, SROWS)], ibuf)
    pltpu.sync_copy(dst_hbm.at[pl.ds(s * SROWS, SROWS)], dbuf)
    pltpu.sync_copy(b23_hbm, bbuf)

    offs, nchunks = _partition(ibuf, dbuf, slist, dlist, c)

    def flush(j, r):
        base = offs[r] + j * CHUNK
        h = pltpu.async_copy(feat_hbm.at[slist.at[pl.ds(base, CHUNK)]],
                             rows, gsem)
        for k in range(CHUNK // 16):
            dstage[pl.ds(k * 16, 16)] = dlist[pl.ds(base + k * 16, 16)]
        h.wait()
        pltpu.sync_copy(rows, acc.at[dstage], add=True)

    def writeout(r):
        @pl.when(s < 4)
        def _():
            lo = c * 5000 + r * QR + s * 125
            albase = (lo // 8) * 8
            delta = lo - albase
            pltpu.sync_copy(ndw_hbm.at[pl.ds(albase, 136)], ndbuf)
            pltpu.sync_copy(noise_hbm.at[pl.ds(albase, 136)], nzbuf)
            for b, n in ((0, CHUNK), (CHUNK, 125 - CHUNK)):
                pltpu.sync_copy(acc.at[pl.ds(s * 125 + b, n)],
                                rows.at[pl.ds(0, n)])

                def fin_row(rr, _):
                    for k in range(OUT_DIM // 16):
                        ndv = ndbuf[delta + b + rr, pl.ds(k * 16, 16)]
                        mv = rows[rr, pl.ds(k * 16, 16)] * ndv + bbuf[k, :]
                        lv = (rows[rr, pl.ds(OUT_DIM + k * 16, 16)] * ndv
                              + bbuf[4 + k, :])
                        zv = (mv + nzbuf[delta + b + rr, pl.ds(k * 16, 16)]
                              * jnp.exp(lv))
                        rows[rr, pl.ds(k * 16, 16)] = mv
                        rows[rr, pl.ds(OUT_DIM + k * 16, 16)] = zv
                        nzbuf[delta + b + rr, pl.ds(k * 16, 16)] = lv
                    return 0
                lax.fori_loop(0, n, fin_row, 0)

                oi = (c * NR + r) * 4 + s
                pltpu.sync_copy(rows.at[pl.ds(0, n), pl.ds(0, OUT_DIM)],
                                mean_hbm.at[oi, pl.ds(b, n)])
                pltpu.sync_copy(
                    rows.at[pl.ds(0, n), pl.ds(OUT_DIM, OUT_DIM)],
                    z_hbm.at[oi, pl.ds(b, n)])
                pltpu.sync_copy(nzbuf.at[pl.ds(delta + b, n)],
                                ls_hbm.at[oi, pl.ds(b, n)])

    for r in range(NR):
        _agg_round(r, s, offs, nchunks, flush, acc, rows, writeout)
        lax.fori_loop(0, 36, fill_zero, 0)


def _norms_body(cnt_ref, x_ref, xn_ref, ns_ref, nd_ref, ndw_ref):
    ns = lax.rsqrt(jnp.maximum(cnt_ref[0], 1.0))
    nd = lax.rsqrt(jnp.maximum(cnt_ref[1], 1.0))
    xn_ref[...] = x_ref[...] * ns[:, :1]
    ns_ref[...] = ns
    nd_ref[...] = nd
    ndw = jnp.broadcast_to(nd[:, :1], (N_NODES, OUT_DIM))
    ndw_ref[...] = jnp.concatenate(
        [ndw, jnp.zeros((16, OUT_DIM), jnp.float32)], axis=0)


def _tc_norms(cnts, x):
    return pl.pallas_call(
        _norms_body,
        out_shape=[
            jax.ShapeDtypeStruct((N_NODES, IN_DIM), jnp.float32),
            jax.ShapeDtypeStruct((N_NODES, 16), jnp.float32),
            jax.ShapeDtypeStruct((N_NODES, 16), jnp.float32),
            jax.ShapeDtypeStruct((N_NODES + 16, OUT_DIM), jnp.float32),
        ],
    )(cnts, x)


def _l1_body(p_ref, nd_ref, ns_ref, w_ref, b_ref, out_ref):
    a = p_ref[...] * nd_ref[:, :1]
    h = jnp.dot(a, w_ref[...], preferred_element_type=jnp.float32,
                precision=lax.Precision.HIGHEST) + b_ref[...]
    out_ref[...] = jnp.maximum(h, 0.0) * ns_ref[:, :1]


def _tc_layer1(p, nd, ns, W1, b1):
    grid = 10
    rb = N_NODES // grid
    return pl.pallas_call(
        _l1_body,
        grid=(grid,),
        in_specs=[
            pl.BlockSpec((rb, IN_DIM), lambda i: (i, 0)),
            pl.BlockSpec((rb, 16), lambda i: (i, 0)),
            pl.BlockSpec((rb, 16), lambda i: (i, 0)),
            pl.BlockSpec((IN_DIM, HID_DIM), lambda i: (0, 0)),
            pl.BlockSpec((1, HID_DIM), lambda i: (0, 0)),
        ],
        out_specs=pl.BlockSpec((rb, HID_DIM), lambda i: (i, 0)),
        out_shape=jax.ShapeDtypeStruct((N_NODES, HID_DIM), jnp.float32),
    )(p, nd, ns, W1, b1.reshape(1, HID_DIM))


def _lin23_body(h_ref, w_ref, out_ref):
    out_ref[...] = jnp.dot(h_ref[...], w_ref[...],
                           preferred_element_type=jnp.float32,
                           precision=lax.Precision.HIGHEST)


def _tc_lin23(h, W23):
    grid = 10
    rb = N_NODES // grid
    return pl.pallas_call(
        _lin23_body,
        grid=(grid,),
        in_specs=[
            pl.BlockSpec((rb, IN_DIM), lambda i: (i, 0)),
            pl.BlockSpec((IN_DIM, 2 * OUT_DIM), lambda i: (0, 0)),
        ],
        out_specs=pl.BlockSpec((rb, 2 * OUT_DIM), lambda i: (i, 0)),
        out_shape=jax.ShapeDtypeStruct((N_NODES, 2 * OUT_DIM), jnp.float32),
    )(h, W23)


def _adj_body(z_ref, adj_ref):
    z3 = z_ref[...].reshape(N_GRAPHS, NPG, OUT_DIM)
    zz = lax.dot_general(z3, z3, (((2,), (2,)), ((0,), (0,))),
                         preferred_element_type=jnp.float32,
                         precision=lax.Precision.HIGHEST)
    adj_ref[...] = jax.nn.sigmoid(zz)


def _tc_adj(z):
    return pl.pallas_call(
        _adj_body,
        out_shape=jax.ShapeDtypeStruct((N_GRAPHS, NPG, NPG), jnp.float32),
    )(z)


def kernel(x, edge_index, noise, W1, b1, W2, b2, W3, b3):
    src = edge_index[0].astype(jnp.int32).reshape(EROWS, 32)
    dst = edge_index[1].astype(jnp.int32).reshape(EROWS, 32)
    noise_p = jnp.concatenate(
        [noise, jnp.zeros((16, OUT_DIM), jnp.float32)], axis=0)
    b23 = jnp.concatenate([b2, b3]).reshape(8, 16)

    cnts = _sc_counts(src, dst).reshape(NC, CNR * CROWS, 16)[:, :N_NODES]
    xn, ns, nd, ndw = _tc_norms(cnts, x)
    p1 = _sc_agg(xn, src, dst).reshape(N_NODES, IN_DIM)
    hn = _tc_layer1(p1, nd, ns, W1, b1)
    g23 = _tc_lin23(hn, jnp.concatenate([W2, W3], axis=1))
    mparts, lparts, zparts = _sc_aggfin(g23, src, dst, ndw, noise_p, b23)
    adj = _tc_adj(zparts.reshape(N_NODES, OUT_DIM))
    mean = mparts.reshape(N_GRAPHS, NPG, OUT_DIM)
    ls = lparts.reshape(N_GRAPHS, NPG, OUT_DIM)
    return adj, mean, ls
